# Initial kernel scaffold; baseline (speedup 1.0000x reference)
import jax
import jax.numpy as jnp
from jax.experimental import pallas as pl


def kernel(input_ids, position_ids, token_type_ids, word_embeddings, dense_W, dense_b, pos_table, type_table, norm_weight, norm_bias):
    raise NotImplementedError("write your pallas kernel here")


# speedup vs baseline: 2.0084x; 2.0084x over previous
import functools

import jax
import jax.numpy as jnp
from jax import lax
from jax.experimental import pallas as pl
from jax.experimental.pallas import tpu as pltpu
from jax.experimental.pallas import tpu_sc as plsc

B, S = 4, 2048
EMB, HID = 128, 512
N = B * S
PAD = 8
R = 256

_NC, _NS = 2, 16
_NW = _NC * _NS
TPW = N // _NW
PCH = 128


def _sc_gather(ids, pids, wtab, ptab):
    mesh = plsc.VectorSubcoreMesh(core_axis_name="c", subcore_axis_name="s")

    @functools.partial(
        pl.kernel,
        mesh=mesh,
        out_type=[
            jax.ShapeDtypeStruct((N + 2 * PAD, EMB), jnp.float32),
            jax.ShapeDtypeStruct((N, HID), jnp.float32),
        ],
        scratch_types=[
            pltpu.VMEM((TPW,), jnp.int32),
            pltpu.VMEM((TPW, EMB), jnp.float32),
            pltpu.VMEM((PCH,), jnp.int32),
            pltpu.VMEM((PCH, HID), jnp.float32),
            pltpu.SemaphoreType.DMA,
        ],
    )
    def k(ids_h, pids_h, wtab_h, ptab_h, emb_o, pos_o, idx_v, rows_v, pidx_v, prow_v, sem):
        wid = lax.axis_index("s") * _NC + lax.axis_index("c")
        t0 = wid * TPW
        pltpu.sync_copy(ids_h.at[pl.ds(t0, TPW)], idx_v)
        pltpu.async_copy(wtab_h.at[idx_v], rows_v, sem).wait()
        pltpu.sync_copy(rows_v, emb_o.at[pl.ds(PAD + t0, TPW)])
        for c in range(TPW // PCH):
            pltpu.sync_copy(pids_h.at[pl.ds(t0 + c * PCH, PCH)], pidx_v)
            pltpu.async_copy(ptab_h.at[pidx_v], prow_v, sem).wait()
            pltpu.sync_copy(prow_v, pos_o.at[pl.ds(t0 + c * PCH, PCH)])

    return k(ids, pids, wtab, ptab)


def _tc_body(emb_ref, pos_ref, ttf_ref, w_ref, b_ref, ttab_ref, nw_ref, nb_ref, o_ref):
    i = pl.program_id(0)
    t0 = i * R
    ext = emb_ref[pl.ds(t0, R + 2 * PAD), :]
    center = ext[PAD:PAD + R, :]
    left = ext[PAD + 1:PAD + 1 + R, :]
    right = ext[PAD - 1:PAD - 1 + R, :]
    srow = lax.rem(t0 + lax.broadcasted_iota(jnp.int32, (R, 1), 0), S)
    left = jnp.where(srow == S - 1, 0.0, left)
    right = jnp.where(srow == 0, 0.0, right)
    w = w_ref[...]
    h = jnp.dot(left, w[0:EMB], preferred_element_type=jnp.float32)
    h = h + jnp.dot(center, w[EMB:2 * EMB], preferred_element_type=jnp.float32)
    h = h + jnp.dot(right, w[2 * EMB:3 * EMB], preferred_element_type=jnp.float32)
    h = h + b_ref[...] + pos_ref[...]
    trow0 = ttab_ref[0:1, :]
    h = h + trow0 + ttf_ref[...] * (ttab_ref[1:2, :] - trow0)
    o_ref[...] = h * nw_ref[...] + nb_ref[...]


def _tc_transform(emb_ext, posemb, ttf, dense_w, dense_b, type_table, norm_w, norm_b):
    return pl.pallas_call(
        _tc_body,
        grid=(N // R,),
        in_specs=[
            pl.BlockSpec((N + 2 * PAD, EMB), lambda i: (0, 0)),
            pl.BlockSpec((R, HID), lambda i: (i, 0)),
            pl.BlockSpec((R, 1), lambda i: (i, 0)),
            pl.BlockSpec((3 * EMB, HID), lambda i: (0, 0)),
            pl.BlockSpec((1, HID), lambda i: (0, 0)),
            pl.BlockSpec((2, HID), lambda i: (0, 0)),
            pl.BlockSpec((1, HID), lambda i: (0, 0)),
            pl.BlockSpec((1, HID), lambda i: (0, 0)),
        ],
        out_specs=pl.BlockSpec((R, HID), lambda i: (i, 0)),
        out_shape=jax.ShapeDtypeStruct((N, HID), jnp.float32),
    )(emb_ext, posemb, ttf, dense_w, dense_b, type_table, norm_w, norm_b)


def kernel(input_ids, position_ids, token_type_ids, word_embeddings, dense_W, dense_b,
           pos_table, type_table, norm_weight, norm_bias):
    ids = input_ids.reshape(-1)
    pids = position_ids.reshape(-1)
    ttf = token_type_ids.reshape(-1, 1).astype(jnp.float32)
    emb_ext, posemb = _sc_gather(ids, pids, word_embeddings, pos_table)
    out = _tc_transform(
        emb_ext, posemb, ttf, dense_W,
        dense_b.reshape(1, HID), type_table,
        norm_weight.reshape(1, HID), norm_bias.reshape(1, HID),
    )
    return out.reshape(B, S, HID)


# speedup vs baseline: 2.0344x; 1.0130x over previous
import functools

import jax
import jax.numpy as jnp
from jax import lax
from jax.experimental import pallas as pl
from jax.experimental.pallas import tpu as pltpu
from jax.experimental.pallas import tpu_sc as plsc

B, S = 4, 2048
EMB, HID = 128, 512
N = B * S
PAD = 8
R = 256

_NC, _NS = 2, 16
_NW = _NC * _NS
TPW = N // _NW
PCH = 64


def _sc_gather(ids, pids2, wtab, ptab):
    mesh = plsc.VectorSubcoreMesh(core_axis_name="c", subcore_axis_name="s")
    NCH = TPW // PCH

    @functools.partial(
        pl.kernel,
        mesh=mesh,
        out_type=[
            jax.ShapeDtypeStruct((N + 2 * PAD, EMB), jnp.float32),
            jax.ShapeDtypeStruct((N, HID), jnp.float32),
        ],
        scratch_types=[
            pltpu.VMEM((TPW,), jnp.int32),
            pltpu.VMEM((NCH, PCH), jnp.int32),
            pltpu.VMEM((TPW, EMB), jnp.float32),
            pltpu.VMEM((PCH, HID), jnp.float32),
            pltpu.VMEM((PCH, HID), jnp.float32),
            pltpu.SemaphoreType.DMA,
            pltpu.SemaphoreType.DMA,
            pltpu.SemaphoreType.DMA,
            pltpu.SemaphoreType.DMA,
            pltpu.SemaphoreType.DMA,
            pltpu.SemaphoreType.DMA,
        ],
    )
    def k(ids_h, pids2_h, wtab_h, ptab_h, emb_o, pos_o,
          idx_v, pidx_v, wrows, pr0, pr1,
          gsw, gs0, gs1, wsw, ws0, ws1):
        wid = lax.axis_index("s") * _NC + lax.axis_index("c")
        t0 = wid * TPW
        pltpu.sync_copy(ids_h.at[pl.ds(t0, TPW)], idx_v)
        pltpu.sync_copy(pids2_h.at[pl.ds(wid * NCH, NCH)], pidx_v)
        cw = pltpu.async_copy(wtab_h.at[idx_v], wrows, gsw)
        g0 = pltpu.async_copy(ptab_h.at[pidx_v.at[0]], pr0, gs0)
        g1 = pltpu.async_copy(ptab_h.at[pidx_v.at[1]], pr1, gs1)
        cw.wait()
        ww = pltpu.async_copy(wrows, emb_o.at[pl.ds(PAD + t0, TPW)], wsw)
        g0.wait()
        w0 = pltpu.async_copy(pr0, pos_o.at[pl.ds(t0, PCH)], ws0)
        g1.wait()
        w1 = pltpu.async_copy(pr1, pos_o.at[pl.ds(t0 + PCH, PCH)], ws1)
        w0.wait()
        g2 = pltpu.async_copy(ptab_h.at[pidx_v.at[2]], pr0, gs0)
        w1.wait()
        g3 = pltpu.async_copy(ptab_h.at[pidx_v.at[3]], pr1, gs1)
        g2.wait()
        w2 = pltpu.async_copy(pr0, pos_o.at[pl.ds(t0 + 2 * PCH, PCH)], ws0)
        g3.wait()
        w3 = pltpu.async_copy(pr1, pos_o.at[pl.ds(t0 + 3 * PCH, PCH)], ws1)
        ww.wait()
        w2.wait()
        w3.wait()

    return k(ids, pids2, wtab, ptab)


def _tc_body(emb_ref, pos_ref, ttf_ref, w_ref, b_ref, ttab_ref, nw_ref, nb_ref, o_ref):
    i = pl.program_id(0)
    t0 = i * R
    ext = emb_ref[pl.ds(t0, R + 2 * PAD), :]
    center = ext[PAD:PAD + R, :]
    left = ext[PAD + 1:PAD + 1 + R, :]
    right = ext[PAD - 1:PAD - 1 + R, :]
    srow = lax.rem(t0 + lax.broadcasted_iota(jnp.int32, (R, 1), 0), S)
    left = jnp.where(srow == S - 1, 0.0, left)
    right = jnp.where(srow == 0, 0.0, right)
    w = w_ref[...]
    h = jnp.dot(left, w[0:EMB], preferred_element_type=jnp.float32)
    h = h + jnp.dot(center, w[EMB:2 * EMB], preferred_element_type=jnp.float32)
    h = h + jnp.dot(right, w[2 * EMB:3 * EMB], preferred_element_type=jnp.float32)
    h = h + b_ref[...] + pos_ref[...]
    trow0 = ttab_ref[0:1, :]
    h = h + trow0 + ttf_ref[...] * (ttab_ref[1:2, :] - trow0)
    o_ref[...] = h * nw_ref[...] + nb_ref[...]


def _tc_transform(emb_ext, posemb, ttf, dense_w, dense_b, type_table, norm_w, norm_b):
    return pl.pallas_call(
        _tc_body,
        grid=(N // R,),
        in_specs=[
            pl.BlockSpec((N + 2 * PAD, EMB), lambda i: (0, 0)),
            pl.BlockSpec((R, HID), lambda i: (i, 0)),
            pl.BlockSpec((R, 1), lambda i: (i, 0)),
            pl.BlockSpec((3 * EMB, HID), lambda i: (0, 0)),
            pl.BlockSpec((1, HID), lambda i: (0, 0)),
            pl.BlockSpec((2, HID), lambda i: (0, 0)),
            pl.BlockSpec((1, HID), lambda i: (0, 0)),
            pl.BlockSpec((1, HID), lambda i: (0, 0)),
        ],
        out_specs=pl.BlockSpec((R, HID), lambda i: (i, 0)),
        out_shape=jax.ShapeDtypeStruct((N, HID), jnp.float32),
    )(emb_ext, posemb, ttf, dense_w, dense_b, type_table, norm_w, norm_b)


def kernel(input_ids, position_ids, token_type_ids, word_embeddings, dense_W, dense_b,
           pos_table, type_table, norm_weight, norm_bias):
    ids = input_ids.reshape(-1)
    pids2 = position_ids.reshape(-1, PCH)
    ttf = token_type_ids.reshape(-1, 1).astype(jnp.float32)
    emb_ext, posemb = _sc_gather(ids, pids2, word_embeddings, pos_table)
    out = _tc_transform(
        emb_ext, posemb, ttf, dense_W,
        dense_b.reshape(1, HID), type_table,
        norm_weight.reshape(1, HID), norm_bias.reshape(1, HID),
    )
    return out.reshape(B, S, HID)


# speedup vs baseline: 2.3099x; 1.1354x over previous
import functools

import jax
import jax.numpy as jnp
from jax import lax
from jax.experimental import pallas as pl
from jax.experimental.pallas import tpu as pltpu
from jax.experimental.pallas import tpu_sc as plsc

B, S = 4, 2048
EMB, HID = 128, 512
N = B * S
PAD = 8
R = 512

_NC, _NS = 2, 16
_NW = _NC * _NS
TPW = N // _NW
PCH = 64


def _sc_gather(ids, pids2, wtab, ptab):
    mesh = plsc.VectorSubcoreMesh(core_axis_name="c", subcore_axis_name="s")
    NCH = TPW // PCH

    @functools.partial(
        pl.kernel,
        mesh=mesh,
        out_type=[
            jax.ShapeDtypeStruct((N + 2 * PAD, EMB), jnp.float32),
            jax.ShapeDtypeStruct((N, HID), jnp.float32),
        ],
        scratch_types=[
            pltpu.VMEM((TPW,), jnp.int32),
            pltpu.VMEM((NCH, PCH), jnp.int32),
            pltpu.VMEM((TPW, EMB), jnp.float32),
            pltpu.VMEM((PCH, HID), jnp.float32),
            pltpu.VMEM((PCH, HID), jnp.float32),
            pltpu.SemaphoreType.DMA,
            pltpu.SemaphoreType.DMA,
            pltpu.SemaphoreType.DMA,
            pltpu.SemaphoreType.DMA,
            pltpu.SemaphoreType.DMA,
            pltpu.SemaphoreType.DMA,
        ],
    )
    def k(ids_h, pids2_h, wtab_h, ptab_h, emb_o, pos_o,
          idx_v, pidx_v, wrows, pr0, pr1,
          gsw, gs0, gs1, wsw, ws0, ws1):
        wid = lax.axis_index("s") * _NC + lax.axis_index("c")
        t0 = wid * TPW
        pltpu.sync_copy(ids_h.at[pl.ds(t0, TPW)], idx_v)
        pltpu.sync_copy(pids2_h.at[pl.ds(wid * NCH, NCH)], pidx_v)
        cw = pltpu.async_copy(wtab_h.at[idx_v], wrows, gsw)
        g0 = pltpu.async_copy(ptab_h.at[pidx_v.at[0]], pr0, gs0)
        g1 = pltpu.async_copy(ptab_h.at[pidx_v.at[1]], pr1, gs1)
        cw.wait()
        ww = pltpu.async_copy(wrows, emb_o.at[pl.ds(PAD + t0, TPW)], wsw)
        g0.wait()
        w0 = pltpu.async_copy(pr0, pos_o.at[pl.ds(t0, PCH)], ws0)
        g1.wait()
        w1 = pltpu.async_copy(pr1, pos_o.at[pl.ds(t0 + PCH, PCH)], ws1)
        w0.wait()
        g2 = pltpu.async_copy(ptab_h.at[pidx_v.at[2]], pr0, gs0)
        w1.wait()
        g3 = pltpu.async_copy(ptab_h.at[pidx_v.at[3]], pr1, gs1)
        g2.wait()
        w2 = pltpu.async_copy(pr0, pos_o.at[pl.ds(t0 + 2 * PCH, PCH)], ws0)
        g3.wait()
        w3 = pltpu.async_copy(pr1, pos_o.at[pl.ds(t0 + 3 * PCH, PCH)], ws1)
        ww.wait()
        w2.wait()
        w3.wait()

    return k(ids, pids2, wtab, ptab)


def _tc_body(emb_ref, pos_ref, ttf_ref, w_ref, b_ref, ttab_ref, nw_ref, nb_ref, o_ref):
    i = pl.program_id(0)
    t0 = i * R
    ext = emb_ref[pl.ds(t0, R + 2 * PAD), :]
    center = ext[PAD:PAD + R, :]
    left = ext[PAD + 1:PAD + 1 + R, :]
    right = ext[PAD - 1:PAD - 1 + R, :]
    srow = lax.rem(t0 + lax.broadcasted_iota(jnp.int32, (R, 1), 0), S)
    left = jnp.where(srow == S - 1, 0.0, left)
    right = jnp.where(srow == 0, 0.0, right)
    w = w_ref[...]
    h = jnp.dot(left, w[0:EMB], preferred_element_type=jnp.float32)
    h = h + jnp.dot(center, w[EMB:2 * EMB], preferred_element_type=jnp.float32)
    h = h + jnp.dot(right, w[2 * EMB:3 * EMB], preferred_element_type=jnp.float32)
    h = h + b_ref[...] + pos_ref[...]
    trow0 = ttab_ref[0:1, :]
    h = h + trow0 + ttf_ref[...] * (ttab_ref[1:2, :] - trow0)
    o_ref[...] = h * nw_ref[...] + nb_ref[...]


def _tc_transform(emb_ext, posemb, ttf, dense_w, dense_b, type_table, norm_w, norm_b):
    return pl.pallas_call(
        _tc_body,
        grid=(N // R,),
        in_specs=[
            pl.BlockSpec((N + 2 * PAD, EMB), lambda i: (0, 0)),
            pl.BlockSpec((R, HID), lambda i: (i, 0)),
            pl.BlockSpec((R, 1), lambda i: (i, 0)),
            pl.BlockSpec((3 * EMB, HID), lambda i: (0, 0)),
            pl.BlockSpec((1, HID), lambda i: (0, 0)),
            pl.BlockSpec((2, HID), lambda i: (0, 0)),
            pl.BlockSpec((1, HID), lambda i: (0, 0)),
            pl.BlockSpec((1, HID), lambda i: (0, 0)),
        ],
        out_specs=pl.BlockSpec((R, HID), lambda i: (i, 0)),
        out_shape=jax.ShapeDtypeStruct((N, HID), jnp.float32),
    )(emb_ext, posemb, ttf, dense_w, dense_b, type_table, norm_w, norm_b)


def kernel(input_ids, position_ids, token_type_ids, word_embeddings, dense_W, dense_b,
           pos_table, type_table, norm_weight, norm_bias):
    ids = input_ids.reshape(-1)
    pids2 = position_ids.reshape(-1, PCH)
    ttf = token_type_ids.reshape(-1, 1).astype(jnp.float32)
    emb_ext, posemb = _sc_gather(ids, pids2, word_embeddings, pos_table)
    out = _tc_transform(
        emb_ext, posemb, ttf, dense_W,
        dense_b.reshape(1, HID), type_table,
        norm_weight.reshape(1, HID), norm_bias.reshape(1, HID),
    )
    return out.reshape(B, S, HID)


# speedup vs baseline: 2.4099x; 1.0433x over previous
import functools

import jax
import jax.numpy as jnp
from jax import lax
from jax.experimental import pallas as pl
from jax.experimental.pallas import tpu as pltpu
from jax.experimental.pallas import tpu_sc as plsc

B, S = 4, 2048
EMB, HID = 128, 512
N = B * S
PAD = 8
R = 1024

_NC, _NS = 2, 16
_NW = _NC * _NS
TPW = N // _NW
PCH = 64


def _sc_gather(ids, pids2, wtab, ptab):
    mesh = plsc.VectorSubcoreMesh(core_axis_name="c", subcore_axis_name="s")
    NCH = TPW // PCH

    @functools.partial(
        pl.kernel,
        mesh=mesh,
        out_type=[
            jax.ShapeDtypeStruct((N + 2 * PAD, EMB), jnp.float32),
            jax.ShapeDtypeStruct((N, HID), jnp.float32),
        ],
        scratch_types=[
            pltpu.VMEM((TPW,), jnp.int32),
            pltpu.VMEM((NCH, PCH), jnp.int32),
            pltpu.VMEM((TPW, EMB), jnp.float32),
            pltpu.VMEM((PCH, HID), jnp.float32),
            pltpu.VMEM((PCH, HID), jnp.float32),
            pltpu.SemaphoreType.DMA,
            pltpu.SemaphoreType.DMA,
            pltpu.SemaphoreType.DMA,
            pltpu.SemaphoreType.DMA,
            pltpu.SemaphoreType.DMA,
            pltpu.SemaphoreType.DMA,
        ],
    )
    def k(ids_h, pids2_h, wtab_h, ptab_h, emb_o, pos_o,
          idx_v, pidx_v, wrows, pr0, pr1,
          gsw, gs0, gs1, wsw, ws0, ws1):
        wid = lax.axis_index("s") * _NC + lax.axis_index("c")
        t0 = wid * TPW
        pltpu.sync_copy(ids_h.at[pl.ds(t0, TPW)], idx_v)
        pltpu.sync_copy(pids2_h.at[pl.ds(wid * NCH, NCH)], pidx_v)
        cw = pltpu.async_copy(wtab_h.at[idx_v], wrows, gsw)
        g0 = pltpu.async_copy(ptab_h.at[pidx_v.at[0]], pr0, gs0)
        g1 = pltpu.async_copy(ptab_h.at[pidx_v.at[1]], pr1, gs1)
        cw.wait()
        ww = pltpu.async_copy(wrows, emb_o.at[pl.ds(PAD + t0, TPW)], wsw)
        g0.wait()
        w0 = pltpu.async_copy(pr0, pos_o.at[pl.ds(t0, PCH)], ws0)
        g1.wait()
        w1 = pltpu.async_copy(pr1, pos_o.at[pl.ds(t0 + PCH, PCH)], ws1)
        w0.wait()
        g2 = pltpu.async_copy(ptab_h.at[pidx_v.at[2]], pr0, gs0)
        w1.wait()
        g3 = pltpu.async_copy(ptab_h.at[pidx_v.at[3]], pr1, gs1)
        g2.wait()
        w2 = pltpu.async_copy(pr0, pos_o.at[pl.ds(t0 + 2 * PCH, PCH)], ws0)
        g3.wait()
        w3 = pltpu.async_copy(pr1, pos_o.at[pl.ds(t0 + 3 * PCH, PCH)], ws1)
        ww.wait()
        w2.wait()
        w3.wait()

    return k(ids, pids2, wtab, ptab)


def _tc_body(emb_ref, pos_ref, ttf_ref, w_ref, b_ref, ttab_ref, nw_ref, nb_ref, o_ref):
    i = pl.program_id(0)
    t0 = i * R
    ext = emb_ref[pl.ds(t0, R + 2 * PAD), :]
    center = ext[PAD:PAD + R, :]
    left = ext[PAD + 1:PAD + 1 + R, :]
    right = ext[PAD - 1:PAD - 1 + R, :]
    srow = lax.rem(t0 + lax.broadcasted_iota(jnp.int32, (R, 1), 0), S)
    left = jnp.where(srow == S - 1, 0.0, left)
    right = jnp.where(srow == 0, 0.0, right)
    w = w_ref[...]
    h = jnp.dot(left, w[0:EMB], preferred_element_type=jnp.float32)
    h = h + jnp.dot(center, w[EMB:2 * EMB], preferred_element_type=jnp.float32)
    h = h + jnp.dot(right, w[2 * EMB:3 * EMB], preferred_element_type=jnp.float32)
    h = h + b_ref[...] + pos_ref[...]
    trow0 = ttab_ref[0:1, :]
    h = h + trow0 + ttf_ref[...] * (ttab_ref[1:2, :] - trow0)
    o_ref[...] = h * nw_ref[...] + nb_ref[...]


def _tc_transform(emb_ext, posemb, ttf, dense_w, dense_b, type_table, norm_w, norm_b):
    return pl.pallas_call(
        _tc_body,
        grid=(N // R,),
        in_specs=[
            pl.BlockSpec((N + 2 * PAD, EMB), lambda i: (0, 0)),
            pl.BlockSpec((R, HID), lambda i: (i, 0)),
            pl.BlockSpec((R, 1), lambda i: (i, 0)),
            pl.BlockSpec((3 * EMB, HID), lambda i: (0, 0)),
            pl.BlockSpec((1, HID), lambda i: (0, 0)),
            pl.BlockSpec((2, HID), lambda i: (0, 0)),
            pl.BlockSpec((1, HID), lambda i: (0, 0)),
            pl.BlockSpec((1, HID), lambda i: (0, 0)),
        ],
        out_specs=pl.BlockSpec((R, HID), lambda i: (i, 0)),
        out_shape=jax.ShapeDtypeStruct((N, HID), jnp.float32),
    )(emb_ext, posemb, ttf, dense_w, dense_b, type_table, norm_w, norm_b)


def kernel(input_ids, position_ids, token_type_ids, word_embeddings, dense_W, dense_b,
           pos_table, type_table, norm_weight, norm_bias):
    ids = input_ids.reshape(-1)
    pids2 = position_ids.reshape(-1, PCH)
    ttf = token_type_ids.reshape(-1, 1).astype(jnp.float32)
    emb_ext, posemb = _sc_gather(ids, pids2, word_embeddings, pos_table)
    out = _tc_transform(
        emb_ext, posemb, ttf, dense_W,
        dense_b.reshape(1, HID), type_table,
        norm_weight.reshape(1, HID), norm_bias.reshape(1, HID),
    )
    return out.reshape(B, S, HID)


# speedup vs baseline: 2.4475x; 1.0156x over previous
import functools

import jax
import jax.numpy as jnp
from jax import lax
from jax.experimental import pallas as pl
from jax.experimental.pallas import tpu as pltpu
from jax.experimental.pallas import tpu_sc as plsc

B, S = 4, 2048
EMB, HID = 128, 512
N = B * S
PAD = 8
R = 1024
WPS = 8

_NC, _NS = 2, 16
_NW = _NC * _NS
TPW = N // _NW
PCH = 64
NCH = TPW // PCH


def _sc_gather(ids, pids, wtab, ptab):
    mesh = plsc.VectorSubcoreMesh(core_axis_name="c", subcore_axis_name="s")

    @functools.partial(
        pl.kernel,
        mesh=mesh,
        out_type=[
            jax.ShapeDtypeStruct((N + 2 * PAD, EMB), jnp.float32),
            jax.ShapeDtypeStruct((N, HID), jnp.float32),
        ],
        scratch_types=[
            pltpu.VMEM((TPW,), jnp.int32),
            pltpu.VMEM((TPW,), jnp.int32),
            pltpu.VMEM((TPW, EMB), jnp.float32),
            pltpu.VMEM((PCH, HID), jnp.float32),
            pltpu.VMEM((PCH, HID), jnp.float32),
            pltpu.SemaphoreType.DMA,
            pltpu.SemaphoreType.DMA,
            pltpu.SemaphoreType.DMA,
            pltpu.SemaphoreType.DMA,
            pltpu.SemaphoreType.DMA,
            pltpu.SemaphoreType.DMA,
        ],
    )
    def k(ids_h, pids_h, wtab_h, ptab_h, emb_o, pos_o,
          idx_v, pidx_v, wrows, pr0, pr1,
          gsw, gs0, gs1, wsw, ws0, ws1):
        wid = lax.axis_index("s") * _NC + lax.axis_index("c")
        b = wid // WPS
        s0 = (wid % WPS) * TPW
        t0 = wid * TPW
        pltpu.sync_copy(ids_h.at[b, pl.ds(s0, TPW)], idx_v)
        pltpu.sync_copy(pids_h.at[b, pl.ds(s0, TPW)], pidx_v)
        cw = pltpu.async_copy(wtab_h.at[idx_v], wrows, gsw)
        g0 = pltpu.async_copy(ptab_h.at[pidx_v.at[pl.ds(0, PCH)]], pr0, gs0)
        g1 = pltpu.async_copy(ptab_h.at[pidx_v.at[pl.ds(PCH, PCH)]], pr1, gs1)
        cw.wait()
        ww = pltpu.async_copy(wrows, emb_o.at[pl.ds(PAD + t0, TPW)], wsw)
        g0.wait()
        w0 = pltpu.async_copy(pr0, pos_o.at[pl.ds(t0, PCH)], ws0)
        g1.wait()
        w1 = pltpu.async_copy(pr1, pos_o.at[pl.ds(t0 + PCH, PCH)], ws1)
        w0.wait()
        g2 = pltpu.async_copy(ptab_h.at[pidx_v.at[pl.ds(2 * PCH, PCH)]], pr0, gs0)
        w1.wait()
        g3 = pltpu.async_copy(ptab_h.at[pidx_v.at[pl.ds(3 * PCH, PCH)]], pr1, gs1)
        g2.wait()
        w2 = pltpu.async_copy(pr0, pos_o.at[pl.ds(t0 + 2 * PCH, PCH)], ws0)
        g3.wait()
        w3 = pltpu.async_copy(pr1, pos_o.at[pl.ds(t0 + 3 * PCH, PCH)], ws1)
        ww.wait()
        w2.wait()
        w3.wait()

    return k(ids, pids, wtab, ptab)


def _tc_body(emb_ref, pos_ref, ttf_ref, w_ref, b_ref, ttab_ref, nw_ref, nb_ref, o_ref):
    bi = pl.program_id(0)
    j = pl.program_id(1)
    t0 = bi * S + j * R
    ext = emb_ref[pl.ds(t0, R + 2 * PAD), :]
    center = ext[PAD:PAD + R, :]
    left = ext[PAD + 1:PAD + 1 + R, :]
    right = ext[PAD - 1:PAD - 1 + R, :]
    srow = j * R + lax.broadcasted_iota(jnp.int32, (R, 1), 0)
    left = jnp.where(srow == S - 1, 0.0, left)
    right = jnp.where(srow == 0, 0.0, right)
    w = w_ref[...]
    h = jnp.dot(left, w[0:EMB], preferred_element_type=jnp.float32)
    h = h + jnp.dot(center, w[EMB:2 * EMB], preferred_element_type=jnp.float32)
    h = h + jnp.dot(right, w[2 * EMB:3 * EMB], preferred_element_type=jnp.float32)
    h = h + b_ref[...] + pos_ref[...]
    trow0 = ttab_ref[0:1, :]
    h = h + trow0 + ttf_ref[...] * (ttab_ref[1:2, :] - trow0)
    o_ref[...] = (h * nw_ref[...] + nb_ref[...])[None]


def _tc_transform(emb_ext, posemb, ttf, dense_w, dense_b, type_table, norm_w, norm_b):
    jpb = S // R
    return pl.pallas_call(
        _tc_body,
        grid=(B, jpb),
        in_specs=[
            pl.BlockSpec((N + 2 * PAD, EMB), lambda bi, j: (0, 0)),
            pl.BlockSpec((R, HID), lambda bi, j: (bi * jpb + j, 0)),
            pl.BlockSpec((R, 1), lambda bi, j: (bi * jpb + j, 0)),
            pl.BlockSpec((3 * EMB, HID), lambda bi, j: (0, 0)),
            pl.BlockSpec((1, HID), lambda bi, j: (0, 0)),
            pl.BlockSpec((2, HID), lambda bi, j: (0, 0)),
            pl.BlockSpec((1, HID), lambda bi, j: (0, 0)),
            pl.BlockSpec((1, HID), lambda bi, j: (0, 0)),
        ],
        out_specs=pl.BlockSpec((1, R, HID), lambda bi, j: (bi, j, 0)),
        out_shape=jax.ShapeDtypeStruct((B, S, HID), jnp.float32),
    )(emb_ext, posemb, ttf, dense_w, dense_b, type_table, norm_w, norm_b)


def kernel(input_ids, position_ids, token_type_ids, word_embeddings, dense_W, dense_b,
           pos_table, type_table, norm_weight, norm_bias):
    ttf = token_type_ids.reshape(-1, 1).astype(jnp.float32)
    emb_ext, posemb = _sc_gather(input_ids, position_ids, word_embeddings, pos_table)
    return _tc_transform(
        emb_ext, posemb, ttf, dense_W,
        dense_b.reshape(1, HID), type_table,
        norm_weight.reshape(1, HID), norm_bias.reshape(1, HID),
    )
